# baseline (speedup 1.0000x reference)
import jax
import jax.numpy as jnp
from jax.experimental import pallas as pl
from jax.experimental.pallas import tpu as pltpu

_GAMMA = 1.5


def _qfocal_body(pred_ref, label_ref, score_ref, out_ref):
    x = pred_ref[...]
    sc = score_ref[...]
    lab = label_ref[...]

    t = jnp.exp2(jnp.minimum(x * -1.4426950408889634, 126.0))
    d1 = 1.0 + t
    sig = 1.0 / d1
    bce0 = jnp.maximum(jnp.log(d1) + x, 0.0)

    s = jnp.max(sc, axis=1, keepdims=True)
    cid = jax.lax.broadcasted_iota(jnp.int32, x.shape, 1)
    m = cid == lab
    a = jnp.where(m, bce0 - x * s, bce0)
    b = jnp.where(m, jnp.abs(s - sig), sig)
    out_ref[...] = a * b * jnp.sqrt(b)


def kernel(pred, label, score):
    B, N, C = pred.shape
    bb = 2
    pt = jnp.transpose(pred, (0, 2, 1))
    st = jnp.transpose(score, (0, 2, 1))
    l3 = label.reshape(B, 1, N)
    bn = N // 2
    out = pl.pallas_call(
        _qfocal_body,
        grid=(B // bb, N // bn),
        in_specs=[
            pl.BlockSpec((bb, C, bn), lambda i, j: (i, 0, j)),
            pl.BlockSpec((bb, 1, bn), lambda i, j: (i, 0, j)),
            pl.BlockSpec((bb, C, bn), lambda i, j: (i, 0, j)),
        ],
        out_specs=pl.BlockSpec((bb, C, bn), lambda i, j: (i, 0, j)),
        out_shape=jax.ShapeDtypeStruct((B, C, N), jnp.float32),
        compiler_params=pltpu.CompilerParams(
            dimension_semantics=("parallel", "parallel"),
        ),
    )(pt, l3, st)
    return jnp.transpose(out, (0, 2, 1))


# speedup vs baseline: 1.0616x; 1.0616x over previous
import jax
import jax.numpy as jnp
from jax.experimental import pallas as pl
from jax.experimental.pallas import tpu as pltpu

_GAMMA = 1.5


def _qfocal_body(pred_ref, label_ref, score_ref, out_ref):
    x = pred_ref[...]
    sc = score_ref[...]
    lab = label_ref[...]

    t = jnp.exp2(jnp.minimum(x * -1.4426950408889634, 126.0))
    d1 = 1.0 + t
    sig = 1.0 / d1
    bce0 = jnp.maximum(jnp.log(d1) + x, 0.0)

    s = jnp.max(sc, axis=1, keepdims=True)
    cid = jax.lax.broadcasted_iota(jnp.int32, x.shape, 1)
    m = cid == lab
    a = jnp.where(m, bce0 - x * s, bce0)
    b = jnp.where(m, jnp.abs(s - sig), sig)
    out_ref[...] = a * b * jnp.sqrt(b)


def kernel(pred, label, score):
    B, N, C = pred.shape
    bb = 2
    pt = jnp.transpose(pred, (0, 2, 1))
    st = jnp.transpose(score, (0, 2, 1))
    l3 = label.reshape(B, 1, N)
    out = pl.pallas_call(
        _qfocal_body,
        grid=(B // bb,),
        in_specs=[
            pl.BlockSpec((bb, C, N), lambda i: (i, 0, 0)),
            pl.BlockSpec((bb, 1, N), lambda i: (i, 0, 0)),
            pl.BlockSpec((bb, C, N), lambda i: (i, 0, 0)),
        ],
        out_specs=pl.BlockSpec((bb, C, N), lambda i: (i, 0, 0)),
        out_shape=jax.ShapeDtypeStruct((B, C, N), jnp.float32),
        compiler_params=pltpu.CompilerParams(
            dimension_semantics=("parallel",),
        ),
    )(pt, l3, st)
    return jnp.transpose(out, (0, 2, 1))
